# baseline (speedup 1.0000x reference)
import functools

import jax
import jax.numpy as jnp
from jax import lax
from jax.experimental import pallas as pl
from jax.experimental.pallas import tpu as pltpu
from jax.experimental.pallas import tpu_sc as plsc

N = 16384
D = 64
F = 4
NC = 2
NS = 16
NW = NC * NS
CPW = N // NW
CHUNK = 128
J = CPW // CHUNK


def _sc_embed(field_a, field_b, field_c, field_d, wa, wb, wc, wd):
    mesh = plsc.VectorSubcoreMesh(core_axis_name="c", subcore_axis_name="s")

    @functools.partial(
        pl.kernel,
        out_type=jax.ShapeDtypeStruct((N, F, D), jnp.float32),
        mesh=mesh,
        scratch_types=[
            pltpu.VMEM((F, J, CHUNK), jnp.int32),
            pltpu.VMEM((2, CHUNK, D), jnp.float32),
            pltpu.SemaphoreType.DMA,
            pltpu.SemaphoreType.DMA,
            pltpu.SemaphoreType.DMA,
        ],
        compiler_params=pltpu.CompilerParams(use_tc_tiling_on_sc=False),
    )
    def k(ia, ib, ic, idd, ta, tb, tc, td, out, idx_v, buf, isem, gsem0, gsem1):
        wid = lax.axis_index("s") * NC + lax.axis_index("c")
        base = wid * CPW
        idx_hbm = (ia, ib, ic, idd)
        tables = (ta, tb, tc, td)

        loads = []
        for f in range(F):
            for j in range(J):
                loads.append(pltpu.async_copy(
                    idx_hbm[f].at[pl.ds(base + j * CHUNK, CHUNK)],
                    idx_v.at[f, j], isem))
        for h in loads:
            h.wait()

        steps = [(f, j) for f in range(F) for j in range(J)]
        gsems = (gsem0, gsem1)

        def start(s, slot):
            f, j = steps[s]
            return pltpu.async_copy(
                tables[f].at[idx_v.at[f, j]], buf.at[slot], gsems[slot])

        pending = start(0, 0)
        for s in range(len(steps)):
            nxt = start(s + 1, (s + 1) % 2) if s + 1 < len(steps) else None
            pending.wait()
            f, j = steps[s]
            pltpu.sync_copy(buf.at[s % 2],
                            out.at[pl.ds(base + j * CHUNK, CHUNK), f])
            pending = nxt

    return k(field_a, field_b, field_c, field_d, wa, wb, wc, wd)


def kernel(field_a, field_b, field_c, field_d,
           W_field_a, W_field_b, W_field_c, W_field_d):
    return _sc_embed(field_a, field_b, field_c, field_d,
                     W_field_a, W_field_b, W_field_c, W_field_d)
